# baseline (speedup 1.0000x reference)
import functools

import jax
import jax.numpy as jnp
from jax import lax
from jax.experimental import pallas as pl
from jax.experimental.pallas import tpu as pltpu
from jax.experimental.pallas import tpu_sc as plsc

BATCH = 16384
EMBED = 32
HIDDEN = 128
PACK = 128 // EMBED
MLP_BLOCK = 2048

NUM_CORES = 2
NUM_SUBCORES = 16
NUM_WORKERS = NUM_CORES * NUM_SUBCORES
PER_WORKER = BATCH // NUM_WORKERS
CHUNK = 128


def _sc_gather(up_hbm_arr, ap_hbm_arr, uid4, aid4):
    mesh = plsc.VectorSubcoreMesh(core_axis_name="c", subcore_axis_name="s")
    out_t = (jax.ShapeDtypeStruct((BATCH, 4 * EMBED), jnp.float32),
             jax.ShapeDtypeStruct((BATCH, 4 * EMBED), jnp.float32))

    @functools.partial(
        pl.kernel, mesh=mesh, out_type=out_t,
        scratch_types=[
            pltpu.VMEM((CHUNK,), jnp.int32),
            pltpu.VMEM((CHUNK,), jnp.int32),
            pltpu.VMEM((CHUNK, 4 * EMBED), jnp.float32),
            pltpu.VMEM((CHUNK, 4 * EMBED), jnp.float32),
            pltpu.SemaphoreType.DMA,
            pltpu.SemaphoreType.DMA,
        ],
    )
    def gather_kernel(ue_hbm, ae_hbm, ui_hbm, ai_hbm, uo_hbm, ao_hbm,
                      uix, aix, urows, arows, usem, asem):
        wid = lax.axis_index("s") * NUM_CORES + lax.axis_index("c")
        base = wid * PER_WORKER

        @pl.loop(0, PER_WORKER, step=CHUNK)
        def _(off):
            b = base + off
            pltpu.sync_copy(ui_hbm.at[pl.ds(b, CHUNK)], uix)
            pltpu.sync_copy(ai_hbm.at[pl.ds(b, CHUNK)], aix)
            cu = pltpu.async_copy(ue_hbm.at[uix], urows, usem)
            ca = pltpu.async_copy(ae_hbm.at[aix], arows, asem)
            cu.wait()
            ca.wait()
            pltpu.sync_copy(urows, uo_hbm.at[pl.ds(b, CHUNK)])
            pltpu.sync_copy(arows, ao_hbm.at[pl.ds(b, CHUNK)])

    return gather_kernel(up_hbm_arr, ap_hbm_arr, uid4, aid4)


def _select_group(rows, ids):
    sel = (ids & (PACK - 1))[:, None]
    out = jnp.where(sel == 0, rows[:, 0:EMBED], 0.0)
    for k in range(1, PACK):
        out = out + jnp.where(sel == k, rows[:, k * EMBED:(k + 1) * EMBED],
                              0.0)
    return out


def _mlp_body(u4_ref, a4_ref, uid_ref, aid_ref, w1u_ref, w1a_ref, b1_ref,
              w2_ref, b2_ref, o_ref):
    u = _select_group(u4_ref[...], uid_ref[...])
    a = _select_group(a4_ref[...], aid_ref[...])
    h = jnp.dot(u, w1u_ref[...], preferred_element_type=jnp.float32)
    h = h + jnp.dot(a, w1a_ref[...], preferred_element_type=jnp.float32)
    h = jnp.maximum(h + b1_ref[...], 0.0)
    o_ref[...] = jnp.sum(h * w2_ref[...], axis=1) + b2_ref[0, 0]


def _mlp(u4, a4, user_ids, anime_ids, W1, b1, W2, b2):
    w1u = W1[:EMBED]
    w1a = W1[EMBED:]
    b1r = b1.reshape(1, HIDDEN)
    w2r = W2.reshape(1, HIDDEN)
    b2r = b2.reshape(1, 1)
    grid = (BATCH // MLP_BLOCK,)
    return pl.pallas_call(
        _mlp_body,
        grid=grid,
        in_specs=[
            pl.BlockSpec((MLP_BLOCK, 4 * EMBED), lambda i: (i, 0)),
            pl.BlockSpec((MLP_BLOCK, 4 * EMBED), lambda i: (i, 0)),
            pl.BlockSpec((MLP_BLOCK,), lambda i: (i,)),
            pl.BlockSpec((MLP_BLOCK,), lambda i: (i,)),
            pl.BlockSpec((EMBED, HIDDEN), lambda i: (0, 0)),
            pl.BlockSpec((EMBED, HIDDEN), lambda i: (0, 0)),
            pl.BlockSpec((1, HIDDEN), lambda i: (0, 0)),
            pl.BlockSpec((1, HIDDEN), lambda i: (0, 0)),
            pl.BlockSpec((1, 1), lambda i: (0, 0)),
        ],
        out_specs=pl.BlockSpec((MLP_BLOCK,), lambda i: (i,)),
        out_shape=jax.ShapeDtypeStruct((BATCH,), jnp.float32),
    )(u4, a4, user_ids, anime_ids, w1u, w1a, b1r, w2r, b2r)


@jax.jit
def kernel(user_ids, anime_ids, user_emb, anime_emb, W1, b1, W2, b2):
    up = user_emb.reshape(-1, 4 * EMBED)
    ap = anime_emb.reshape(-1, 4 * EMBED)
    uid4 = lax.shift_right_logical(user_ids, 2)
    aid4 = lax.shift_right_logical(anime_ids, 2)
    u4, a4 = _sc_gather(up, ap, uid4, aid4)
    return _mlp(u4, a4, user_ids, anime_ids, W1, b1, W2, b2)


# speedup vs baseline: 1.6662x; 1.6662x over previous
import functools

import jax
import jax.numpy as jnp
from jax import lax
from jax.experimental import pallas as pl
from jax.experimental.pallas import tpu as pltpu
from jax.experimental.pallas import tpu_sc as plsc

BATCH = 16384
EMBED = 32
HIDDEN = 128
PACK = 128 // EMBED
W = 2048
MLP_BLOCK = 2048

NUM_CORES = 2
NUM_SUBCORES = 16
NUM_WORKERS = NUM_CORES * NUM_SUBCORES
PER_WORKER = BATCH // NUM_WORKERS
CHUNK = 128


def _transpose_pack_body(x0_ref, x1_ref, x2_ref, x3_ref, o_ref):
    o_ref[:, 0 * EMBED:1 * EMBED] = x0_ref[...].T
    o_ref[:, 1 * EMBED:2 * EMBED] = x1_ref[...].T
    o_ref[:, 2 * EMBED:3 * EMBED] = x2_ref[...].T
    o_ref[:, 3 * EMBED:4 * EMBED] = x3_ref[...].T


def _transpose_pack(table_t, n_rows):
    grid_n = pl.cdiv(n_rows, PACK * W)
    out_rows = grid_n * W
    last_block = n_rows // W

    def mk_spec(k):
        return pl.BlockSpec(
            (EMBED, W),
            lambda i, k=k: (0, jnp.minimum(PACK * i + k, last_block)))

    return pl.pallas_call(
        _transpose_pack_body,
        grid=(grid_n,),
        in_specs=[mk_spec(0), mk_spec(1), mk_spec(2), mk_spec(3)],
        out_specs=pl.BlockSpec((W, PACK * EMBED), lambda i: (i, 0)),
        out_shape=jax.ShapeDtypeStruct((out_rows, PACK * EMBED), jnp.float32),
    )(table_t, table_t, table_t, table_t)


def _sc_gather(up, ap, uid_pk, aid_pk):
    mesh = plsc.VectorSubcoreMesh(core_axis_name="c", subcore_axis_name="s")
    out_t = (jax.ShapeDtypeStruct((BATCH, PACK * EMBED), jnp.float32),
             jax.ShapeDtypeStruct((BATCH, PACK * EMBED), jnp.float32))

    @functools.partial(
        pl.kernel, mesh=mesh, out_type=out_t,
        scratch_types=[
            pltpu.VMEM((CHUNK,), jnp.int32),
            pltpu.VMEM((CHUNK,), jnp.int32),
            pltpu.VMEM((CHUNK, PACK * EMBED), jnp.float32),
            pltpu.VMEM((CHUNK, PACK * EMBED), jnp.float32),
            pltpu.SemaphoreType.DMA,
            pltpu.SemaphoreType.DMA,
        ],
    )
    def gather_kernel(ue_hbm, ae_hbm, ui_hbm, ai_hbm, uo_hbm, ao_hbm,
                      uix, aix, urows, arows, usem, asem):
        wid = lax.axis_index("s") * NUM_CORES + lax.axis_index("c")
        base = wid * PER_WORKER

        @pl.loop(0, PER_WORKER, step=CHUNK)
        def _(off):
            b = base + off
            pltpu.sync_copy(ui_hbm.at[pl.ds(b, CHUNK)], uix)
            pltpu.sync_copy(ai_hbm.at[pl.ds(b, CHUNK)], aix)
            cu = pltpu.async_copy(ue_hbm.at[uix], urows, usem)
            ca = pltpu.async_copy(ae_hbm.at[aix], arows, asem)
            cu.wait()
            ca.wait()
            pltpu.sync_copy(urows, uo_hbm.at[pl.ds(b, CHUNK)])
            pltpu.sync_copy(arows, ao_hbm.at[pl.ds(b, CHUNK)])

    return gather_kernel(up, ap, uid_pk, aid_pk)


def _select_group(rows, sel):
    sel = sel[:, None]
    out = jnp.where(sel == 0, rows[:, 0:EMBED], 0.0)
    for k in range(1, PACK):
        out = out + jnp.where(sel == k, rows[:, k * EMBED:(k + 1) * EMBED],
                              0.0)
    return out


def _mlp_body(u4_ref, a4_ref, usel_ref, asel_ref, w1u_ref, w1a_ref, b1_ref,
              w2_ref, b2_ref, o_ref):
    u = _select_group(u4_ref[...], usel_ref[...])
    a = _select_group(a4_ref[...], asel_ref[...])
    h = jnp.dot(u, w1u_ref[...], preferred_element_type=jnp.float32)
    h = h + jnp.dot(a, w1a_ref[...], preferred_element_type=jnp.float32)
    h = jnp.maximum(h + b1_ref[...], 0.0)
    o_ref[...] = jnp.sum(h * w2_ref[...], axis=1) + b2_ref[0, 0]


def _mlp(u4, a4, usel, asel, W1, b1, W2, b2):
    w1u = W1[:EMBED]
    w1a = W1[EMBED:]
    b1r = b1.reshape(1, HIDDEN)
    w2r = W2.reshape(1, HIDDEN)
    b2r = b2.reshape(1, 1)
    grid = (BATCH // MLP_BLOCK,)
    return pl.pallas_call(
        _mlp_body,
        grid=grid,
        in_specs=[
            pl.BlockSpec((MLP_BLOCK, PACK * EMBED), lambda i: (i, 0)),
            pl.BlockSpec((MLP_BLOCK, PACK * EMBED), lambda i: (i, 0)),
            pl.BlockSpec((MLP_BLOCK,), lambda i: (i,)),
            pl.BlockSpec((MLP_BLOCK,), lambda i: (i,)),
            pl.BlockSpec((EMBED, HIDDEN), lambda i: (0, 0)),
            pl.BlockSpec((EMBED, HIDDEN), lambda i: (0, 0)),
            pl.BlockSpec((1, HIDDEN), lambda i: (0, 0)),
            pl.BlockSpec((1, HIDDEN), lambda i: (0, 0)),
            pl.BlockSpec((1, 1), lambda i: (0, 0)),
        ],
        out_specs=pl.BlockSpec((MLP_BLOCK,), lambda i: (i,)),
        out_shape=jax.ShapeDtypeStruct((BATCH,), jnp.float32),
    )(u4, a4, usel, asel, w1u, w1a, b1r, w2r, b2r)


def _packed_index(ids):
    i = ids // (PACK * W)
    rem = ids % (PACK * W)
    sel = rem // W
    p = rem % W
    return i * W + p, sel


@jax.jit
def kernel(user_ids, anime_ids, user_emb, anime_emb, W1, b1, W2, b2):
    ap = _transpose_pack(anime_emb.T, anime_emb.shape[0])
    up = _transpose_pack(user_emb.T, user_emb.shape[0])
    uid_pk, usel = _packed_index(user_ids)
    aid_pk, asel = _packed_index(anime_ids)
    u4, a4 = _sc_gather(up, ap, uid_pk, aid_pk)
    return _mlp(u4, a4, usel, asel, W1, b1, W2, b2)


# speedup vs baseline: 2.0697x; 1.2422x over previous
import functools

import jax
import jax.numpy as jnp
from jax import lax
from jax.experimental import pallas as pl
from jax.experimental.pallas import tpu as pltpu
from jax.experimental.pallas import tpu_sc as plsc

BATCH = 16384
EMBED = 32
HIDDEN = 128
PACK = 128 // EMBED
W = 2048
MLP_BLOCK = 2048

NUM_CORES = 2
NUM_SUBCORES = 16
NUM_WORKERS = NUM_CORES * NUM_SUBCORES
PER_WORKER = BATCH // NUM_WORKERS
CHUNK = 128


def _transpose_pack_body(x0_ref, x1_ref, x2_ref, x3_ref, e_ref, o_ref):
    xs = (x0_ref, x1_ref, x2_ref, x3_ref)
    acc = None
    for k in range(PACK):
        ek = e_ref[pl.ds(k * EMBED, EMBED), :]
        xk = xs[k][...]
        hi = xk.astype(jnp.bfloat16)
        lo = (xk - hi.astype(jnp.float32)).astype(jnp.bfloat16)
        ekb = ek.astype(jnp.bfloat16)
        dims = (((0,), (0,)), ((), ()))
        t = lax.dot_general(hi, ekb, dims,
                            preferred_element_type=jnp.float32)
        t = t + lax.dot_general(lo, ekb, dims,
                                preferred_element_type=jnp.float32)
        acc = t if acc is None else acc + t
    o_ref[...] = acc


def _transpose_pack(table_t, eye_pack, n_rows):
    grid_n = pl.cdiv(n_rows, PACK * W)
    out_rows = grid_n * W
    last_block = n_rows // W

    def mk_spec(k):
        return pl.BlockSpec(
            (EMBED, W),
            lambda i, k=k: (0, jnp.minimum(PACK * i + k, last_block)))

    return pl.pallas_call(
        _transpose_pack_body,
        grid=(grid_n,),
        in_specs=[mk_spec(0), mk_spec(1), mk_spec(2), mk_spec(3),
                  pl.BlockSpec((PACK * EMBED, PACK * EMBED),
                               lambda i: (0, 0))],
        out_specs=pl.BlockSpec((W, PACK * EMBED), lambda i: (i, 0)),
        out_shape=jax.ShapeDtypeStruct((out_rows, PACK * EMBED), jnp.float32),
    )(table_t, table_t, table_t, table_t, eye_pack)


def _sc_gather(up, ap, uid_pk, aid_pk):
    mesh = plsc.VectorSubcoreMesh(core_axis_name="c", subcore_axis_name="s")
    out_t = (jax.ShapeDtypeStruct((BATCH, PACK * EMBED), jnp.float32),
             jax.ShapeDtypeStruct((BATCH, PACK * EMBED), jnp.float32))

    @functools.partial(
        pl.kernel, mesh=mesh, out_type=out_t,
        scratch_types=[
            pltpu.VMEM((CHUNK,), jnp.int32),
            pltpu.VMEM((CHUNK,), jnp.int32),
            pltpu.VMEM((CHUNK, PACK * EMBED), jnp.float32),
            pltpu.VMEM((CHUNK, PACK * EMBED), jnp.float32),
            pltpu.SemaphoreType.DMA,
            pltpu.SemaphoreType.DMA,
        ],
    )
    def gather_kernel(ue_hbm, ae_hbm, ui_hbm, ai_hbm, uo_hbm, ao_hbm,
                      uix, aix, urows, arows, usem, asem):
        wid = lax.axis_index("s") * NUM_CORES + lax.axis_index("c")
        base = wid * PER_WORKER

        @pl.loop(0, PER_WORKER, step=CHUNK)
        def _(off):
            b = base + off
            pltpu.sync_copy(ui_hbm.at[pl.ds(b, CHUNK)], uix)
            pltpu.sync_copy(ai_hbm.at[pl.ds(b, CHUNK)], aix)
            cu = pltpu.async_copy(ue_hbm.at[uix], urows, usem)
            ca = pltpu.async_copy(ae_hbm.at[aix], arows, asem)
            cu.wait()
            ca.wait()
            pltpu.sync_copy(urows, uo_hbm.at[pl.ds(b, CHUNK)])
            pltpu.sync_copy(arows, ao_hbm.at[pl.ds(b, CHUNK)])

    return gather_kernel(up, ap, uid_pk, aid_pk)


def _select_group(rows, sel):
    sel = sel[:, None]
    out = jnp.where(sel == 0, rows[:, 0:EMBED], 0.0)
    for k in range(1, PACK):
        out = out + jnp.where(sel == k, rows[:, k * EMBED:(k + 1) * EMBED],
                              0.0)
    return out


def _mlp_body(u4_ref, a4_ref, usel_ref, asel_ref, w1u_ref, w1a_ref, b1_ref,
              w2_ref, b2_ref, o_ref):
    u = _select_group(u4_ref[...], usel_ref[...])
    a = _select_group(a4_ref[...], asel_ref[...])
    h = jnp.dot(u, w1u_ref[...], preferred_element_type=jnp.float32)
    h = h + jnp.dot(a, w1a_ref[...], preferred_element_type=jnp.float32)
    h = jnp.maximum(h + b1_ref[...], 0.0)
    o_ref[...] = jnp.sum(h * w2_ref[...], axis=1) + b2_ref[0, 0]


def _mlp(u4, a4, usel, asel, W1, b1, W2, b2):
    w1u = W1[:EMBED]
    w1a = W1[EMBED:]
    b1r = b1.reshape(1, HIDDEN)
    w2r = W2.reshape(1, HIDDEN)
    b2r = b2.reshape(1, 1)
    grid = (BATCH // MLP_BLOCK,)
    return pl.pallas_call(
        _mlp_body,
        grid=grid,
        in_specs=[
            pl.BlockSpec((MLP_BLOCK, PACK * EMBED), lambda i: (i, 0)),
            pl.BlockSpec((MLP_BLOCK, PACK * EMBED), lambda i: (i, 0)),
            pl.BlockSpec((MLP_BLOCK,), lambda i: (i,)),
            pl.BlockSpec((MLP_BLOCK,), lambda i: (i,)),
            pl.BlockSpec((EMBED, HIDDEN), lambda i: (0, 0)),
            pl.BlockSpec((EMBED, HIDDEN), lambda i: (0, 0)),
            pl.BlockSpec((1, HIDDEN), lambda i: (0, 0)),
            pl.BlockSpec((1, HIDDEN), lambda i: (0, 0)),
            pl.BlockSpec((1, 1), lambda i: (0, 0)),
        ],
        out_specs=pl.BlockSpec((MLP_BLOCK,), lambda i: (i,)),
        out_shape=jax.ShapeDtypeStruct((BATCH,), jnp.float32),
    )(u4, a4, usel, asel, w1u, w1a, b1r, w2r, b2r)


def _packed_index(ids):
    i = ids // (PACK * W)
    rem = ids % (PACK * W)
    sel = rem // W
    p = rem % W
    return i * W + p, sel


@jax.jit
def kernel(user_ids, anime_ids, user_emb, anime_emb, W1, b1, W2, b2):
    eye_pack = jnp.eye(PACK * EMBED, dtype=jnp.float32)
    ap = _transpose_pack(anime_emb.T, eye_pack, anime_emb.shape[0])
    up = _transpose_pack(user_emb.T, eye_pack, user_emb.shape[0])
    uid_pk, usel = _packed_index(user_ids)
    aid_pk, asel = _packed_index(anime_ids)
    u4, a4 = _sc_gather(up, ap, uid_pk, aid_pk)
    return _mlp(u4, a4, usel, asel, W1, b1, W2, b2)


# speedup vs baseline: 2.4302x; 1.1742x over previous
import functools

import jax
import jax.numpy as jnp
from jax import lax
from jax.experimental import pallas as pl
from jax.experimental.pallas import tpu as pltpu
from jax.experimental.pallas import tpu_sc as plsc

BATCH = 16384
EMBED = 32
HIDDEN = 128
PACK = 128 // EMBED
W = 8192
MLP_BLOCK = 2048

NUM_CORES = 2
NUM_SUBCORES = 16
NUM_WORKERS = NUM_CORES * NUM_SUBCORES
PER_WORKER = BATCH // NUM_WORKERS
CHUNK = 128


def _transpose_pack_body(x0_ref, x1_ref, x2_ref, x3_ref, e_ref, o_ref):
    xs = (x0_ref, x1_ref, x2_ref, x3_ref)
    acc = None
    for k in range(PACK):
        ek = e_ref[pl.ds(k * EMBED, EMBED), :]
        xk = xs[k][...]
        hi = xk.astype(jnp.bfloat16)
        lo = (xk - hi.astype(jnp.float32)).astype(jnp.bfloat16)
        ekb = ek.astype(jnp.bfloat16)
        dims = (((0,), (0,)), ((), ()))
        t = lax.dot_general(hi, ekb, dims,
                            preferred_element_type=jnp.float32)
        t = t + lax.dot_general(lo, ekb, dims,
                                preferred_element_type=jnp.float32)
        acc = t if acc is None else acc + t
    o_ref[...] = acc


def _transpose_pack(table_t, eye_pack, n_rows):
    grid_n = pl.cdiv(n_rows, PACK * W)
    out_rows = grid_n * W
    last_block = n_rows // W

    def mk_spec(k):
        return pl.BlockSpec(
            (EMBED, W),
            lambda i, k=k: (0, jnp.minimum(PACK * i + k, last_block)))

    return pl.pallas_call(
        _transpose_pack_body,
        grid=(grid_n,),
        in_specs=[mk_spec(0), mk_spec(1), mk_spec(2), mk_spec(3),
                  pl.BlockSpec((PACK * EMBED, PACK * EMBED),
                               lambda i: (0, 0))],
        out_specs=pl.BlockSpec((W, PACK * EMBED), lambda i: (i, 0)),
        out_shape=jax.ShapeDtypeStruct((out_rows, PACK * EMBED), jnp.float32),
    )(table_t, table_t, table_t, table_t, eye_pack)


def _sc_gather(up, ap, uid_pk, aid_pk):
    mesh = plsc.VectorSubcoreMesh(core_axis_name="c", subcore_axis_name="s")
    out_t = (jax.ShapeDtypeStruct((BATCH, PACK * EMBED), jnp.float32),
             jax.ShapeDtypeStruct((BATCH, PACK * EMBED), jnp.float32))

    @functools.partial(
        pl.kernel, mesh=mesh, out_type=out_t,
        scratch_types=[
            pltpu.VMEM((CHUNK,), jnp.int32),
            pltpu.VMEM((CHUNK,), jnp.int32),
            pltpu.VMEM((CHUNK, PACK * EMBED), jnp.float32),
            pltpu.VMEM((CHUNK, PACK * EMBED), jnp.float32),
            pltpu.SemaphoreType.DMA,
            pltpu.SemaphoreType.DMA,
        ],
    )
    def gather_kernel(ue_hbm, ae_hbm, ui_hbm, ai_hbm, uo_hbm, ao_hbm,
                      uix, aix, urows, arows, usem, asem):
        wid = lax.axis_index("s") * NUM_CORES + lax.axis_index("c")
        base = wid * PER_WORKER

        @pl.loop(0, PER_WORKER, step=CHUNK)
        def _(off):
            b = base + off
            pltpu.sync_copy(ui_hbm.at[pl.ds(b, CHUNK)], uix)
            pltpu.sync_copy(ai_hbm.at[pl.ds(b, CHUNK)], aix)
            cu = pltpu.async_copy(ue_hbm.at[uix], urows, usem)
            ca = pltpu.async_copy(ae_hbm.at[aix], arows, asem)
            cu.wait()
            ca.wait()
            pltpu.sync_copy(urows, uo_hbm.at[pl.ds(b, CHUNK)])
            pltpu.sync_copy(arows, ao_hbm.at[pl.ds(b, CHUNK)])

    return gather_kernel(up, ap, uid_pk, aid_pk)


def _select_group(rows, sel):
    sel = sel[:, None]
    out = jnp.where(sel == 0, rows[:, 0:EMBED], 0.0)
    for k in range(1, PACK):
        out = out + jnp.where(sel == k, rows[:, k * EMBED:(k + 1) * EMBED],
                              0.0)
    return out


def _mlp_body(u4_ref, a4_ref, usel_ref, asel_ref, w1u_ref, w1a_ref, b1_ref,
              w2_ref, b2_ref, o_ref):
    u = _select_group(u4_ref[...], usel_ref[...])
    a = _select_group(a4_ref[...], asel_ref[...])
    h = jnp.dot(u, w1u_ref[...], preferred_element_type=jnp.float32)
    h = h + jnp.dot(a, w1a_ref[...], preferred_element_type=jnp.float32)
    h = jnp.maximum(h + b1_ref[...], 0.0)
    o_ref[...] = jnp.sum(h * w2_ref[...], axis=1) + b2_ref[0, 0]


def _mlp(u4, a4, usel, asel, W1, b1, W2, b2):
    w1u = W1[:EMBED]
    w1a = W1[EMBED:]
    b1r = b1.reshape(1, HIDDEN)
    w2r = W2.reshape(1, HIDDEN)
    b2r = b2.reshape(1, 1)
    grid = (BATCH // MLP_BLOCK,)
    return pl.pallas_call(
        _mlp_body,
        grid=grid,
        in_specs=[
            pl.BlockSpec((MLP_BLOCK, PACK * EMBED), lambda i: (i, 0)),
            pl.BlockSpec((MLP_BLOCK, PACK * EMBED), lambda i: (i, 0)),
            pl.BlockSpec((MLP_BLOCK,), lambda i: (i,)),
            pl.BlockSpec((MLP_BLOCK,), lambda i: (i,)),
            pl.BlockSpec((EMBED, HIDDEN), lambda i: (0, 0)),
            pl.BlockSpec((EMBED, HIDDEN), lambda i: (0, 0)),
            pl.BlockSpec((1, HIDDEN), lambda i: (0, 0)),
            pl.BlockSpec((1, HIDDEN), lambda i: (0, 0)),
            pl.BlockSpec((1, 1), lambda i: (0, 0)),
        ],
        out_specs=pl.BlockSpec((MLP_BLOCK,), lambda i: (i,)),
        out_shape=jax.ShapeDtypeStruct((BATCH,), jnp.float32),
    )(u4, a4, usel, asel, w1u, w1a, b1r, w2r, b2r)


def _packed_index(ids):
    i = ids // (PACK * W)
    rem = ids % (PACK * W)
    sel = rem // W
    p = rem % W
    return i * W + p, sel


@jax.jit
def kernel(user_ids, anime_ids, user_emb, anime_emb, W1, b1, W2, b2):
    eye_pack = jnp.eye(PACK * EMBED, dtype=jnp.float32)
    ap = _transpose_pack(anime_emb.T, eye_pack, anime_emb.shape[0])
    up = _transpose_pack(user_emb.T, eye_pack, user_emb.shape[0])
    uid_pk, usel = _packed_index(user_ids)
    aid_pk, asel = _packed_index(anime_ids)
    u4, a4 = _sc_gather(up, ap, uid_pk, aid_pk)
    return _mlp(u4, a4, usel, asel, W1, b1, W2, b2)


# speedup vs baseline: 3.5449x; 1.4587x over previous
import functools

import jax
import jax.numpy as jnp
from jax import lax
from jax.experimental import pallas as pl
from jax.experimental.pallas import tpu as pltpu
from jax.experimental.pallas import tpu_sc as plsc

BATCH = 16384
EMBED = 32
HIDDEN = 128
PACK = 128 // EMBED
W = 8192
MLP_BLOCK = 2048

NUM_CORES = 2
NUM_SUBCORES = 16
NUM_WORKERS = NUM_CORES * NUM_SUBCORES
PER_WORKER = BATCH // NUM_WORKERS
CHUNK = 128


def _transpose_pack_body(x0_ref, x1_ref, x2_ref, x3_ref, e_ref, o_ref):
    his, los = [], []
    for xr in (x0_ref, x1_ref, x2_ref, x3_ref):
        xk = xr[...]
        hi = xk.astype(jnp.bfloat16)
        lo = (xk - hi.astype(jnp.float32)).astype(jnp.bfloat16)
        his.append(hi)
        los.append(lo)
    x_all = jnp.concatenate(his + los, axis=0)
    o_ref[...] = lax.dot_general(x_all, e_ref[...],
                                 (((0,), (0,)), ((), ())),
                                 preferred_element_type=jnp.float32)


def _transpose_pack(table_t, eye_pack, n_rows):
    grid_n = pl.cdiv(n_rows, PACK * W)
    out_rows = grid_n * W
    last_block = n_rows // W

    def mk_spec(k):
        return pl.BlockSpec(
            (EMBED, W),
            lambda i, k=k: (0, jnp.minimum(PACK * i + k, last_block)))

    return pl.pallas_call(
        _transpose_pack_body,
        grid=(grid_n,),
        in_specs=[mk_spec(0), mk_spec(1), mk_spec(2), mk_spec(3),
                  pl.BlockSpec((2 * PACK * EMBED, PACK * EMBED),
                               lambda i: (0, 0))],
        out_specs=pl.BlockSpec((W, PACK * EMBED), lambda i: (i, 0)),
        out_shape=jax.ShapeDtypeStruct((out_rows, PACK * EMBED), jnp.float32),
    )(table_t, table_t, table_t, table_t, eye_pack)


def _sc_gather(up, ap, uid_pk, aid_pk):
    mesh = plsc.VectorSubcoreMesh(core_axis_name="c", subcore_axis_name="s")
    out_t = (jax.ShapeDtypeStruct((BATCH, PACK * EMBED), jnp.float32),
             jax.ShapeDtypeStruct((BATCH, PACK * EMBED), jnp.float32))

    @functools.partial(
        pl.kernel, mesh=mesh, out_type=out_t,
        scratch_types=[
            pltpu.VMEM((CHUNK,), jnp.int32),
            pltpu.VMEM((CHUNK,), jnp.int32),
            pltpu.VMEM((CHUNK, PACK * EMBED), jnp.float32),
            pltpu.VMEM((CHUNK, PACK * EMBED), jnp.float32),
            pltpu.SemaphoreType.DMA,
            pltpu.SemaphoreType.DMA,
        ],
    )
    def gather_kernel(ue_hbm, ae_hbm, ui_hbm, ai_hbm, uo_hbm, ao_hbm,
                      uix, aix, urows, arows, usem, asem):
        wid = lax.axis_index("s") * NUM_CORES + lax.axis_index("c")
        base = wid * PER_WORKER

        @pl.loop(0, PER_WORKER, step=CHUNK)
        def _(off):
            b = base + off
            pltpu.sync_copy(ui_hbm.at[pl.ds(b, CHUNK)], uix)
            pltpu.sync_copy(ai_hbm.at[pl.ds(b, CHUNK)], aix)
            cu = pltpu.async_copy(ue_hbm.at[uix], urows, usem)
            ca = pltpu.async_copy(ae_hbm.at[aix], arows, asem)
            cu.wait()
            ca.wait()
            pltpu.sync_copy(urows, uo_hbm.at[pl.ds(b, CHUNK)])
            pltpu.sync_copy(arows, ao_hbm.at[pl.ds(b, CHUNK)])

    return gather_kernel(up, ap, uid_pk, aid_pk)


def _select_group(rows, sel):
    sel = sel[:, None]
    out = jnp.where(sel == 0, rows[:, 0:EMBED], 0.0)
    for k in range(1, PACK):
        out = out + jnp.where(sel == k, rows[:, k * EMBED:(k + 1) * EMBED],
                              0.0)
    return out


def _mlp_body(u4_ref, a4_ref, usel_ref, asel_ref, w1u_ref, w1a_ref, b1_ref,
              w2_ref, b2_ref, o_ref):
    u = _select_group(u4_ref[...], usel_ref[...])
    a = _select_group(a4_ref[...], asel_ref[...])
    h = jnp.dot(u, w1u_ref[...], preferred_element_type=jnp.float32)
    h = h + jnp.dot(a, w1a_ref[...], preferred_element_type=jnp.float32)
    h = jnp.maximum(h + b1_ref[...], 0.0)
    o_ref[...] = jnp.sum(h * w2_ref[...], axis=1) + b2_ref[0, 0]


def _mlp(u4, a4, usel, asel, W1, b1, W2, b2):
    w1u = W1[:EMBED]
    w1a = W1[EMBED:]
    b1r = b1.reshape(1, HIDDEN)
    w2r = W2.reshape(1, HIDDEN)
    b2r = b2.reshape(1, 1)
    grid = (BATCH // MLP_BLOCK,)
    return pl.pallas_call(
        _mlp_body,
        grid=grid,
        in_specs=[
            pl.BlockSpec((MLP_BLOCK, PACK * EMBED), lambda i: (i, 0)),
            pl.BlockSpec((MLP_BLOCK, PACK * EMBED), lambda i: (i, 0)),
            pl.BlockSpec((MLP_BLOCK,), lambda i: (i,)),
            pl.BlockSpec((MLP_BLOCK,), lambda i: (i,)),
            pl.BlockSpec((EMBED, HIDDEN), lambda i: (0, 0)),
            pl.BlockSpec((EMBED, HIDDEN), lambda i: (0, 0)),
            pl.BlockSpec((1, HIDDEN), lambda i: (0, 0)),
            pl.BlockSpec((1, HIDDEN), lambda i: (0, 0)),
            pl.BlockSpec((1, 1), lambda i: (0, 0)),
        ],
        out_specs=pl.BlockSpec((MLP_BLOCK,), lambda i: (i,)),
        out_shape=jax.ShapeDtypeStruct((BATCH,), jnp.float32),
    )(u4, a4, usel, asel, w1u, w1a, b1r, w2r, b2r)


def _packed_index(ids):
    i = ids // (PACK * W)
    rem = ids % (PACK * W)
    sel = rem // W
    p = rem % W
    return i * W + p, sel


@jax.jit
def kernel(user_ids, anime_ids, user_emb, anime_emb, W1, b1, W2, b2):
    eye1 = jnp.eye(PACK * EMBED, dtype=jnp.bfloat16)
    eye_pack = jnp.concatenate([eye1, eye1], axis=0)
    ap = _transpose_pack(anime_emb.T, eye_pack, anime_emb.shape[0])
    up = _transpose_pack(user_emb.T, eye_pack, user_emb.shape[0])
    uid_pk, usel = _packed_index(user_ids)
    aid_pk, asel = _packed_index(anime_ids)
    u4, a4 = _sc_gather(up, ap, uid_pk, aid_pk)
    return _mlp(u4, a4, usel, asel, W1, b1, W2, b2)


# speedup vs baseline: 3.8287x; 1.0801x over previous
import functools

import jax
import jax.numpy as jnp
from jax import lax
from jax.experimental import pallas as pl
from jax.experimental.pallas import tpu as pltpu
from jax.experimental.pallas import tpu_sc as plsc

BATCH = 16384
EMBED = 32
HIDDEN = 128
PACK = 128 // EMBED
W = 8192
MLP_BLOCK = 2048

NUM_CORES = 2
NUM_SUBCORES = 16
NUM_WORKERS = NUM_CORES * NUM_SUBCORES
PER_WORKER = BATCH // NUM_WORKERS
CHUNK = 128


def _transpose_pack_body(x0_ref, x1_ref, x2_ref, x3_ref, e_ref, o_ref):
    his, los = [], []
    for xr in (x0_ref, x1_ref, x2_ref, x3_ref):
        xk = xr[...]
        hi = xk.astype(jnp.bfloat16)
        lo = (xk - hi.astype(jnp.float32)).astype(jnp.bfloat16)
        his.append(hi)
        los.append(lo)
    x_all = jnp.concatenate(his + los, axis=0)
    o_ref[...] = lax.dot_general(x_all, e_ref[...],
                                 (((0,), (0,)), ((), ())),
                                 preferred_element_type=jnp.float32)


def _transpose_pack(table_t, eye_pack, n_rows):
    grid_n = pl.cdiv(n_rows, PACK * W)
    out_rows = grid_n * W
    last_block = n_rows // W

    def mk_spec(k):
        return pl.BlockSpec(
            (EMBED, W),
            lambda i, k=k: (0, jnp.minimum(PACK * i + k, last_block)))

    return pl.pallas_call(
        _transpose_pack_body,
        grid=(grid_n,),
        in_specs=[mk_spec(0), mk_spec(1), mk_spec(2), mk_spec(3),
                  pl.BlockSpec((2 * PACK * EMBED, PACK * EMBED),
                               lambda i: (0, 0))],
        out_specs=pl.BlockSpec((W, PACK * EMBED), lambda i: (i, 0)),
        out_shape=jax.ShapeDtypeStruct((out_rows, PACK * EMBED), jnp.float32),
    )(table_t, table_t, table_t, table_t, eye_pack)


def _sc_gather(up, ap, uid_pk, aid_pk):
    mesh = plsc.VectorSubcoreMesh(core_axis_name="c", subcore_axis_name="s")
    out_t = (jax.ShapeDtypeStruct((BATCH, PACK * EMBED), jnp.float32),
             jax.ShapeDtypeStruct((BATCH, PACK * EMBED), jnp.float32))

    @functools.partial(
        pl.kernel, mesh=mesh, out_type=out_t,
        scratch_types=[
            pltpu.VMEM((CHUNK,), jnp.int32),
            pltpu.VMEM((CHUNK,), jnp.int32),
            pltpu.VMEM((CHUNK, PACK * EMBED), jnp.float32),
            pltpu.VMEM((CHUNK, PACK * EMBED), jnp.float32),
            pltpu.SemaphoreType.DMA,
            pltpu.SemaphoreType.DMA,
        ],
    )
    def gather_kernel(ue_hbm, ae_hbm, ui_hbm, ai_hbm, uo_hbm, ao_hbm,
                      uix, aix, urows, arows, usem, asem):
        wid = lax.axis_index("s") * NUM_CORES + lax.axis_index("c")
        base = wid * PER_WORKER

        @pl.loop(0, PER_WORKER, step=CHUNK)
        def _(off):
            b = base + off
            pltpu.sync_copy(ui_hbm.at[pl.ds(b, CHUNK)], uix)
            pltpu.sync_copy(ai_hbm.at[pl.ds(b, CHUNK)], aix)
            cu = pltpu.async_copy(ue_hbm.at[uix], urows, usem)
            ca = pltpu.async_copy(ae_hbm.at[aix], arows, asem)
            cu.wait()
            ca.wait()
            pltpu.sync_copy(urows, uo_hbm.at[pl.ds(b, CHUNK)])
            pltpu.sync_copy(arows, ao_hbm.at[pl.ds(b, CHUNK)])

    return gather_kernel(up, ap, uid_pk, aid_pk)


_SEL_SHIFT = (W - 1).bit_length()


def _mask_group(rows, ids):
    sel = (ids >> _SEL_SHIFT) & (PACK - 1)
    lane_group = jax.lax.broadcasted_iota(
        jnp.int32, (1, PACK * EMBED), 1) // EMBED
    return jnp.where(lane_group == sel[:, None], rows, 0.0)


def _mlp_body(u4_ref, a4_ref, uid_ref, aid_ref, w_all_ref, b1_ref,
              w2_ref, b2_ref, o_ref):
    um = _mask_group(u4_ref[...], uid_ref[...])
    am = _mask_group(a4_ref[...], aid_ref[...])
    x = jnp.concatenate([um, am], axis=1)
    h = jnp.dot(x, w_all_ref[...], preferred_element_type=jnp.float32)
    h = jnp.maximum(h + b1_ref[...], 0.0)
    o_ref[...] = jnp.sum(h * w2_ref[...], axis=1) + b2_ref[0, 0]


def _mlp(u4, a4, user_ids, anime_ids, W1, b1, W2, b2):
    w_all = jnp.concatenate([jnp.tile(W1[:EMBED], (PACK, 1)),
                             jnp.tile(W1[EMBED:], (PACK, 1))], axis=0)
    b1r = b1.reshape(1, HIDDEN)
    w2r = W2.reshape(1, HIDDEN)
    b2r = b2.reshape(1, 1)
    grid = (BATCH // MLP_BLOCK,)
    return pl.pallas_call(
        _mlp_body,
        grid=grid,
        in_specs=[
            pl.BlockSpec((MLP_BLOCK, PACK * EMBED), lambda i: (i, 0)),
            pl.BlockSpec((MLP_BLOCK, PACK * EMBED), lambda i: (i, 0)),
            pl.BlockSpec((MLP_BLOCK,), lambda i: (i,)),
            pl.BlockSpec((MLP_BLOCK,), lambda i: (i,)),
            pl.BlockSpec((2 * PACK * EMBED, HIDDEN), lambda i: (0, 0)),
            pl.BlockSpec((1, HIDDEN), lambda i: (0, 0)),
            pl.BlockSpec((1, HIDDEN), lambda i: (0, 0)),
            pl.BlockSpec((1, 1), lambda i: (0, 0)),
        ],
        out_specs=pl.BlockSpec((MLP_BLOCK,), lambda i: (i,)),
        out_shape=jax.ShapeDtypeStruct((BATCH,), jnp.float32),
    )(u4, a4, user_ids, anime_ids, w_all, b1r, w2r, b2r)


def _packed_index(ids):
    i = ids // (PACK * W)
    rem = ids % (PACK * W)
    sel = rem // W
    p = rem % W
    return i * W + p, sel


@jax.jit
def kernel(user_ids, anime_ids, user_emb, anime_emb, W1, b1, W2, b2):
    eye1 = jnp.eye(PACK * EMBED, dtype=jnp.bfloat16)
    eye_pack = jnp.concatenate([eye1, eye1], axis=0)
    ap = _transpose_pack(anime_emb.T, eye_pack, anime_emb.shape[0])
    up = _transpose_pack(user_emb.T, eye_pack, user_emb.shape[0])
    uid_pk, _ = _packed_index(user_ids)
    aid_pk, _ = _packed_index(anime_ids)
    u4, a4 = _sc_gather(up, ap, uid_pk, aid_pk)
    return _mlp(u4, a4, user_ids, anime_ids, W1, b1, W2, b2)


# speedup vs baseline: 3.8669x; 1.0100x over previous
import functools

import jax
import jax.numpy as jnp
from jax import lax
from jax.experimental import pallas as pl
from jax.experimental.pallas import tpu as pltpu
from jax.experimental.pallas import tpu_sc as plsc

BATCH = 16384
EMBED = 32
HIDDEN = 128
PACK = 128 // EMBED
W = 8192
MLP_BLOCK = 2048

NUM_CORES = 2
NUM_SUBCORES = 16
NUM_WORKERS = NUM_CORES * NUM_SUBCORES
PER_WORKER = BATCH // NUM_WORKERS
CHUNK = 128


def _transpose_pack_body(x0_ref, x1_ref, x2_ref, x3_ref, e_ref, o_ref):
    his, los = [], []
    for xr in (x0_ref, x1_ref, x2_ref, x3_ref):
        xk = xr[...]
        hi = xk.astype(jnp.bfloat16)
        lo = (xk - hi.astype(jnp.float32)).astype(jnp.bfloat16)
        his.append(hi)
        los.append(lo)
    x_all = jnp.concatenate(his + los, axis=0)
    o_ref[...] = lax.dot_general(x_all, e_ref[...],
                                 (((0,), (0,)), ((), ())),
                                 preferred_element_type=jnp.float32)


def _transpose_pack(table_t, eye_pack, n_rows):
    grid_n = pl.cdiv(n_rows, PACK * W)
    out_rows = grid_n * W
    last_block = n_rows // W

    def mk_spec(k):
        return pl.BlockSpec(
            (EMBED, W),
            lambda i, k=k: (0, jnp.minimum(PACK * i + k, last_block)))

    return pl.pallas_call(
        _transpose_pack_body,
        grid=(grid_n,),
        in_specs=[mk_spec(0), mk_spec(1), mk_spec(2), mk_spec(3),
                  pl.BlockSpec((2 * PACK * EMBED, PACK * EMBED),
                               lambda i: (0, 0))],
        out_specs=pl.BlockSpec((W, PACK * EMBED), lambda i: (i, 0)),
        out_shape=jax.ShapeDtypeStruct((out_rows, PACK * EMBED), jnp.float32),
    )(table_t, table_t, table_t, table_t, eye_pack)


def _sc_gather(table, ids):
    mesh = plsc.VectorSubcoreMesh(core_axis_name="c", subcore_axis_name="s")
    out_t = jax.ShapeDtypeStruct((BATCH, PACK * EMBED), jnp.float32)

    @functools.partial(
        pl.kernel, mesh=mesh, out_type=out_t,
        scratch_types=[
            pltpu.VMEM((CHUNK,), jnp.int32),
            pltpu.VMEM((CHUNK, PACK * EMBED), jnp.float32),
            pltpu.SemaphoreType.DMA,
        ],
    )
    def gather_kernel(e_hbm, i_hbm, o_hbm, ix, rows, sem):
        wid = lax.axis_index("s") * NUM_CORES + lax.axis_index("c")
        base = wid * PER_WORKER

        @pl.loop(0, PER_WORKER, step=CHUNK)
        def _(off):
            b = base + off
            pltpu.sync_copy(i_hbm.at[pl.ds(b, CHUNK)], ix)
            for j in range(CHUNK // 16):
                sl = pl.ds(j * 16, 16)
                v = ix[sl]
                ix[sl] = ((v >> (_SEL_SHIFT + 2)) << _SEL_SHIFT) | (v & (W - 1))
            pltpu.async_copy(e_hbm.at[ix], rows, sem).wait()
            pltpu.sync_copy(rows, o_hbm.at[pl.ds(b, CHUNK)])

    return gather_kernel(table, ids)


_SEL_SHIFT = (W - 1).bit_length()


def _mask_group(rows, ids):
    sel = (ids >> _SEL_SHIFT) & (PACK - 1)
    lane_group = jax.lax.broadcasted_iota(
        jnp.int32, (1, PACK * EMBED), 1) // EMBED
    return jnp.where(lane_group == sel[:, None], rows, 0.0)


def _mlp_body(u4_ref, a4_ref, uid_ref, aid_ref, w1_ref, b1_ref,
              w2_ref, b2_ref, o_ref):
    um = _mask_group(u4_ref[...], uid_ref[...])
    am = _mask_group(a4_ref[...], aid_ref[...])
    x = jnp.concatenate([um, am], axis=1)
    w1u = w1_ref[:EMBED, :]
    w1a = w1_ref[EMBED:, :]
    w_all = jnp.concatenate([w1u] * PACK + [w1a] * PACK, axis=0)
    h = jnp.dot(x, w_all, preferred_element_type=jnp.float32)
    h = jnp.maximum(h + b1_ref[...], 0.0)
    o_ref[...] = jnp.sum(h * w2_ref[...], axis=1) + b2_ref[0, 0]


def _mlp(u4, a4, user_ids, anime_ids, W1, b1, W2, b2):
    b1r = b1.reshape(1, HIDDEN)
    w2r = W2.reshape(1, HIDDEN)
    b2r = b2.reshape(1, 1)
    grid = (BATCH // MLP_BLOCK,)
    return pl.pallas_call(
        _mlp_body,
        grid=grid,
        in_specs=[
            pl.BlockSpec((MLP_BLOCK, PACK * EMBED), lambda i: (i, 0)),
            pl.BlockSpec((MLP_BLOCK, PACK * EMBED), lambda i: (i, 0)),
            pl.BlockSpec((MLP_BLOCK,), lambda i: (i,)),
            pl.BlockSpec((MLP_BLOCK,), lambda i: (i,)),
            pl.BlockSpec((2 * EMBED, HIDDEN), lambda i: (0, 0)),
            pl.BlockSpec((1, HIDDEN), lambda i: (0, 0)),
            pl.BlockSpec((1, HIDDEN), lambda i: (0, 0)),
            pl.BlockSpec((1, 1), lambda i: (0, 0)),
        ],
        out_specs=pl.BlockSpec((MLP_BLOCK,), lambda i: (i,)),
        out_shape=jax.ShapeDtypeStruct((BATCH,), jnp.float32),
    )(u4, a4, user_ids, anime_ids, W1, b1r, w2r, b2r)


@jax.jit
def kernel(user_ids, anime_ids, user_emb, anime_emb, W1, b1, W2, b2):
    eye1 = jnp.eye(PACK * EMBED, dtype=jnp.bfloat16)
    eye_pack = jnp.concatenate([eye1, eye1], axis=0)
    ap = _transpose_pack(anime_emb.T, eye_pack, anime_emb.shape[0])
    a4 = _sc_gather(ap, anime_ids)
    up = _transpose_pack(user_emb.T, eye_pack, user_emb.shape[0])
    u4 = _sc_gather(up, user_ids)
    return _mlp(u4, a4, user_ids, anime_ids, W1, b1, W2, b2)


# speedup vs baseline: 3.8705x; 1.0009x over previous
import functools

import jax
import jax.numpy as jnp
from jax import lax
from jax.experimental import pallas as pl
from jax.experimental.pallas import tpu as pltpu
from jax.experimental.pallas import tpu_sc as plsc

BATCH = 16384
EMBED = 32
HIDDEN = 128
PACK = 128 // EMBED
W = 8192
MLP_BLOCK = 4096

NUM_CORES = 2
NUM_SUBCORES = 16
NUM_WORKERS = NUM_CORES * NUM_SUBCORES
PER_WORKER = BATCH // NUM_WORKERS
CHUNK = 128


def _transpose_pack_body(x0_ref, x1_ref, x2_ref, x3_ref, e_ref, o_ref):
    his, los = [], []
    for xr in (x0_ref, x1_ref, x2_ref, x3_ref):
        xk = xr[...]
        hi = xk.astype(jnp.bfloat16)
        lo = (xk - hi.astype(jnp.float32)).astype(jnp.bfloat16)
        his.append(hi)
        los.append(lo)
    x_all = jnp.concatenate(his + los, axis=0)
    o_ref[...] = lax.dot_general(x_all, e_ref[...],
                                 (((0,), (0,)), ((), ())),
                                 preferred_element_type=jnp.float32)


def _transpose_pack(table_t, eye_pack, n_rows):
    grid_n = pl.cdiv(n_rows, PACK * W)
    out_rows = grid_n * W
    last_block = n_rows // W

    def mk_spec(k):
        return pl.BlockSpec(
            (EMBED, W),
            lambda i, k=k: (0, jnp.minimum(PACK * i + k, last_block)))

    return pl.pallas_call(
        _transpose_pack_body,
        grid=(grid_n,),
        in_specs=[mk_spec(0), mk_spec(1), mk_spec(2), mk_spec(3),
                  pl.BlockSpec((2 * PACK * EMBED, PACK * EMBED),
                               lambda i: (0, 0))],
        out_specs=pl.BlockSpec((W, PACK * EMBED), lambda i: (i, 0)),
        out_shape=jax.ShapeDtypeStruct((out_rows, PACK * EMBED), jnp.float32),
    )(table_t, table_t, table_t, table_t, eye_pack)


def _sc_gather(table, ids):
    mesh = plsc.VectorSubcoreMesh(core_axis_name="c", subcore_axis_name="s")
    out_t = jax.ShapeDtypeStruct((BATCH, PACK * EMBED), jnp.float32)

    @functools.partial(
        pl.kernel, mesh=mesh, out_type=out_t,
        scratch_types=[
            pltpu.VMEM((CHUNK,), jnp.int32),
            pltpu.VMEM((CHUNK, PACK * EMBED), jnp.float32),
            pltpu.SemaphoreType.DMA,
        ],
    )
    def gather_kernel(e_hbm, i_hbm, o_hbm, ix, rows, sem):
        wid = lax.axis_index("s") * NUM_CORES + lax.axis_index("c")
        base = wid * PER_WORKER

        @pl.loop(0, PER_WORKER, step=CHUNK)
        def _(off):
            b = base + off
            pltpu.sync_copy(i_hbm.at[pl.ds(b, CHUNK)], ix)
            for j in range(CHUNK // 16):
                sl = pl.ds(j * 16, 16)
                v = ix[sl]
                ix[sl] = ((v >> (_SEL_SHIFT + 2)) << _SEL_SHIFT) | (v & (W - 1))
            pltpu.async_copy(e_hbm.at[ix], rows, sem).wait()
            pltpu.sync_copy(rows, o_hbm.at[pl.ds(b, CHUNK)])

    return gather_kernel(table, ids)


_SEL_SHIFT = (W - 1).bit_length()


def _mask_group(rows, ids):
    sel = (ids >> _SEL_SHIFT) & (PACK - 1)
    lane_group = jax.lax.broadcasted_iota(
        jnp.int32, (1, PACK * EMBED), 1) // EMBED
    return jnp.where(lane_group == sel[:, None], rows, 0.0)


def _mlp_u_body(u4_ref, uid_ref, w1_ref, b1_ref, h_ref):
    um = _mask_group(u4_ref[...], uid_ref[...])
    w_u = jnp.concatenate([w1_ref[:EMBED, :]] * PACK, axis=0)
    h_ref[...] = jnp.dot(um, w_u,
                         preferred_element_type=jnp.float32) + b1_ref[...]


def _mlp_fin_body(a4_ref, aid_ref, hu_ref, w1_ref, w2_ref, b2_ref, o_ref):
    am = _mask_group(a4_ref[...], aid_ref[...])
    w_a = jnp.concatenate([w1_ref[EMBED:, :]] * PACK, axis=0)
    h = hu_ref[...] + jnp.dot(am, w_a, preferred_element_type=jnp.float32)
    h = jnp.maximum(h, 0.0)
    o_ref[...] = jnp.sum(h * w2_ref[...], axis=1) + b2_ref[0, 0]


def _mlp_u(u4, user_ids, W1, b1):
    b1r = b1.reshape(1, HIDDEN)
    grid = (BATCH // MLP_BLOCK,)
    return pl.pallas_call(
        _mlp_u_body,
        grid=grid,
        in_specs=[
            pl.BlockSpec((MLP_BLOCK, PACK * EMBED), lambda i: (i, 0)),
            pl.BlockSpec((MLP_BLOCK,), lambda i: (i,)),
            pl.BlockSpec((2 * EMBED, HIDDEN), lambda i: (0, 0)),
            pl.BlockSpec((1, HIDDEN), lambda i: (0, 0)),
        ],
        out_specs=pl.BlockSpec((MLP_BLOCK, HIDDEN), lambda i: (i, 0)),
        out_shape=jax.ShapeDtypeStruct((BATCH, HIDDEN), jnp.float32),
    )(u4, user_ids, W1, b1r)


def _mlp_fin(a4, anime_ids, hu, W1, W2, b2):
    w2r = W2.reshape(1, HIDDEN)
    b2r = b2.reshape(1, 1)
    grid = (BATCH // MLP_BLOCK,)
    return pl.pallas_call(
        _mlp_fin_body,
        grid=grid,
        in_specs=[
            pl.BlockSpec((MLP_BLOCK, PACK * EMBED), lambda i: (i, 0)),
            pl.BlockSpec((MLP_BLOCK,), lambda i: (i,)),
            pl.BlockSpec((MLP_BLOCK, HIDDEN), lambda i: (i, 0)),
            pl.BlockSpec((2 * EMBED, HIDDEN), lambda i: (0, 0)),
            pl.BlockSpec((1, HIDDEN), lambda i: (0, 0)),
            pl.BlockSpec((1, 1), lambda i: (0, 0)),
        ],
        out_specs=pl.BlockSpec((MLP_BLOCK,), lambda i: (i,)),
        out_shape=jax.ShapeDtypeStruct((BATCH,), jnp.float32),
    )(a4, anime_ids, hu, W1, w2r, b2r)


@jax.jit
def kernel(user_ids, anime_ids, user_emb, anime_emb, W1, b1, W2, b2):
    eye1 = jnp.eye(PACK * EMBED, dtype=jnp.bfloat16)
    eye_pack = jnp.concatenate([eye1, eye1], axis=0)
    up = _transpose_pack(user_emb.T, eye_pack, user_emb.shape[0])
    u4 = _sc_gather(up, user_ids)
    ap = _transpose_pack(anime_emb.T, eye_pack, anime_emb.shape[0])
    a4 = _sc_gather(ap, anime_ids)
    hu = _mlp_u(u4, user_ids, W1, b1)
    return _mlp_fin(a4, anime_ids, hu, W1, W2, b2)


# speedup vs baseline: 4.0028x; 1.0342x over previous
import functools

import jax
import jax.numpy as jnp
from jax import lax
from jax.experimental import pallas as pl
from jax.experimental.pallas import tpu as pltpu
from jax.experimental.pallas import tpu_sc as plsc

BATCH = 16384
EMBED = 32
HIDDEN = 128
PACK = 128 // EMBED
W = 8192
MLP_BLOCK = 4096

NUM_CORES = 2
NUM_SUBCORES = 16
NUM_WORKERS = NUM_CORES * NUM_SUBCORES
PER_WORKER = BATCH // NUM_WORKERS
CHUNK = 128


def _transpose_pack_body(x0_ref, x1_ref, x2_ref, x3_ref, e_ref, o_ref):
    his, los = [], []
    for xr in (x0_ref, x1_ref, x2_ref, x3_ref):
        xk = xr[...]
        hi = xk.astype(jnp.bfloat16)
        lo = (xk - hi.astype(jnp.float32)).astype(jnp.bfloat16)
        his.append(hi)
        los.append(lo)
    x_all = jnp.concatenate(his + los, axis=0)
    o_ref[...] = lax.dot_general(x_all, e_ref[...],
                                 (((0,), (0,)), ((), ())),
                                 preferred_element_type=jnp.float32)


def _transpose_pack(table_t, eye_pack, n_rows):
    grid_n = pl.cdiv(n_rows, PACK * W)
    out_rows = grid_n * W
    last_block = n_rows // W

    def mk_spec(k):
        return pl.BlockSpec(
            (EMBED, W),
            lambda i, k=k: (0, jnp.minimum(PACK * i + k, last_block)))

    return pl.pallas_call(
        _transpose_pack_body,
        grid=(grid_n,),
        in_specs=[mk_spec(0), mk_spec(1), mk_spec(2), mk_spec(3),
                  pl.BlockSpec((2 * PACK * EMBED, PACK * EMBED),
                               lambda i: (0, 0))],
        out_specs=pl.BlockSpec((W, PACK * EMBED), lambda i: (i, 0)),
        out_shape=jax.ShapeDtypeStruct((out_rows, PACK * EMBED), jnp.float32),
    )(table_t, table_t, table_t, table_t, eye_pack)


def _sc_gather(table, ids):
    mesh = plsc.VectorSubcoreMesh(core_axis_name="c", subcore_axis_name="s")
    out_t = jax.ShapeDtypeStruct((BATCH, PACK * EMBED), jnp.float32)

    n_chunks = PER_WORKER // CHUNK

    @functools.partial(
        pl.kernel, mesh=mesh, out_type=out_t,
        scratch_types=[
            pltpu.VMEM((PER_WORKER,), jnp.int32),
            pltpu.VMEM((CHUNK, PACK * EMBED), jnp.float32),
            pltpu.VMEM((CHUNK, PACK * EMBED), jnp.float32),
            pltpu.SemaphoreType.DMA,
            pltpu.SemaphoreType.DMA,
            pltpu.SemaphoreType.DMA,
            pltpu.SemaphoreType.DMA,
        ],
    )
    def gather_kernel(e_hbm, i_hbm, o_hbm, ix, rows0, rows1,
                      gs0, gs1, ws0, ws1):
        wid = lax.axis_index("s") * NUM_CORES + lax.axis_index("c")
        base = wid * PER_WORKER
        pltpu.sync_copy(i_hbm.at[pl.ds(base, PER_WORKER)], ix)
        for j in range(PER_WORKER // 16):
            sl = pl.ds(j * 16, 16)
            v = ix[sl]
            ix[sl] = ((v >> (_SEL_SHIFT + 2)) << _SEL_SHIFT) | (v & (W - 1))

        bufs = (rows0, rows1)
        gsems = (gs0, gs1)
        wsems = (ws0, ws1)

        def start_gather(c):
            return pltpu.async_copy(
                e_hbm.at[ix.at[pl.ds(c * CHUNK, CHUNK)]], bufs[c % 2],
                gsems[c % 2])

        def start_write(c):
            return pltpu.async_copy(
                bufs[c % 2], o_hbm.at[pl.ds(base + c * CHUNK, CHUNK)],
                wsems[c % 2])

        gathers = [start_gather(0), start_gather(1)]
        writes = [None, None]
        for c in range(n_chunks):
            gathers[c % 2].wait()
            writes[c % 2] = start_write(c)
            nxt = c + 2
            if nxt < n_chunks:
                writes[c % 2].wait()
                gathers[c % 2] = start_gather(nxt)
        for c in (n_chunks - 2, n_chunks - 1):
            writes[c % 2].wait()

    return gather_kernel(table, ids)


_SEL_SHIFT = (W - 1).bit_length()


def _mask_group(rows, ids):
    sel = (ids >> _SEL_SHIFT) & (PACK - 1)
    lane_group = jax.lax.broadcasted_iota(
        jnp.int32, (1, PACK * EMBED), 1) // EMBED
    return jnp.where(lane_group == sel[:, None], rows, 0.0)


def _mlp_body(u4_ref, a4_ref, uid_ref, aid_ref, w1_ref, b1_ref,
              w2_ref, b2_ref, o_ref):
    um = _mask_group(u4_ref[...], uid_ref[...])
    am = _mask_group(a4_ref[...], aid_ref[...])
    x = jnp.concatenate([um, am], axis=1)
    w_all = jnp.concatenate([w1_ref[:EMBED, :]] * PACK
                            + [w1_ref[EMBED:, :]] * PACK, axis=0)
    h = jnp.dot(x, w_all, preferred_element_type=jnp.float32)
    h = jnp.maximum(h + b1_ref[...], 0.0)
    o_ref[...] = jnp.sum(h * w2_ref[...], axis=1) + b2_ref[0, 0]


def _mlp(u4, a4, user_ids, anime_ids, W1, b1, W2, b2):
    b1r = b1.reshape(1, HIDDEN)
    w2r = W2.reshape(1, HIDDEN)
    b2r = b2.reshape(1, 1)
    grid = (BATCH // MLP_BLOCK,)
    return pl.pallas_call(
        _mlp_body,
        grid=grid,
        in_specs=[
            pl.BlockSpec((MLP_BLOCK, PACK * EMBED), lambda i: (i, 0)),
            pl.BlockSpec((MLP_BLOCK, PACK * EMBED), lambda i: (i, 0)),
            pl.BlockSpec((MLP_BLOCK,), lambda i: (i,)),
            pl.BlockSpec((MLP_BLOCK,), lambda i: (i,)),
            pl.BlockSpec((2 * EMBED, HIDDEN), lambda i: (0, 0)),
            pl.BlockSpec((1, HIDDEN), lambda i: (0, 0)),
            pl.BlockSpec((1, HIDDEN), lambda i: (0, 0)),
            pl.BlockSpec((1, 1), lambda i: (0, 0)),
        ],
        out_specs=pl.BlockSpec((MLP_BLOCK,), lambda i: (i,)),
        out_shape=jax.ShapeDtypeStruct((BATCH,), jnp.float32),
    )(u4, a4, user_ids, anime_ids, W1, b1r, w2r, b2r)


@jax.jit
def kernel(user_ids, anime_ids, user_emb, anime_emb, W1, b1, W2, b2):
    eye1 = jnp.eye(PACK * EMBED, dtype=jnp.bfloat16)
    eye_pack = jnp.concatenate([eye1, eye1], axis=0)
    up = _transpose_pack(user_emb.T, eye_pack, user_emb.shape[0])
    u4 = _sc_gather(up, user_ids)
    ap = _transpose_pack(anime_emb.T, eye_pack, anime_emb.shape[0])
    a4 = _sc_gather(ap, anime_ids)
    return _mlp(u4, a4, user_ids, anime_ids, W1, b1, W2, b2)
